# baseline (speedup 1.0000x reference)
import jax
import jax.numpy as jnp
from jax import lax
from jax.experimental import pallas as pl
from jax.experimental.pallas import tpu as pltpu
from jax.experimental.pallas import tpu_sc as plsc

NUM_CORES = 2
NUM_SUBCORES = 16
LANES = 16
NW = NUM_CORES * NUM_SUBCORES
B = 16384
E = 64
N = 100000
PACKC = 16384
BPW = B // NW
HALF = BPW // 2
CHUNK = 128
NCHUNK = BPW // CHUNK

_SC_MESH = dict(core_axis_name="c", subcore_axis_name="s",
                num_cores=NUM_CORES, num_subcores=NUM_SUBCORES)
_SC_PARAMS = pltpu.CompilerParams(use_tc_tiling_on_sc=False)


def _pack_body(ut_ref, mt_ref, out_ref):
    x = jnp.concatenate([ut_ref[...], mt_ref[...]], axis=0)
    ii = lax.broadcasted_iota(jnp.int32, (2 * E, 2 * E), 0)
    jj = lax.broadcasted_iota(jnp.int32, (2 * E, 2 * E), 1)
    eye = (ii == jj).astype(jnp.float32)
    out_ref[...] = lax.dot_general(
        x, eye, dimension_numbers=(((0,), (0,)), ((), ())),
        preferred_element_type=jnp.float32)


def _pack_call(uembT, membT):
    grid = (N + PACKC - 1) // PACKC
    return pl.pallas_call(
        _pack_body,
        grid=(grid,),
        in_specs=[
            pl.BlockSpec((E, PACKC), lambda i: (0, i)),
            pl.BlockSpec((E, PACKC), lambda i: (0, i)),
        ],
        out_specs=pl.BlockSpec((PACKC, 2 * E), lambda i: (i, 0)),
        out_shape=jax.ShapeDtypeStruct((N, 2 * E), jnp.float32),
    )(uembT, membT)


def _sc_body(uidx_hbm, midx_hbm, packed_hbm, bias_hbm,
             partials_hbm, bsum_hbm,
             uidx_v, midx_v, midxb_v, urows0_v, mrows0_v, urows1_v, mrows1_v,
             ub_v, mb_v, bsum_v, pacc_v,
             sem0, sem1, bias_sem):
    wid = lax.axis_index("s") * NUM_CORES + lax.axis_index("c")
    base = wid * BPW

    pltpu.sync_copy(uidx_hbm.at[pl.ds(wid * NCHUNK, NCHUNK)], uidx_v)
    pltpu.sync_copy(midx_hbm.at[pl.ds(wid * NCHUNK, NCHUNK)], midx_v)

    for j in range(NCHUNK):
        for hh in range(CHUNK // LANES):
            sl = pl.ds(hh * LANES, LANES)
            midxb_v[j, sl] = midx_v[j, sl] + N
    bias_copies = []
    for j in range(NCHUNK):
        sl = pl.ds(j * CHUNK, CHUNK)
        bias_copies.append(pltpu.async_copy(
            bias_hbm.at[uidx_v.at[j]], ub_v.at[sl], bias_sem))
        bias_copies.append(pltpu.async_copy(
            bias_hbm.at[midxb_v.at[j]], mb_v.at[sl], bias_sem))

    bufs = ((urows0_v, mrows0_v, sem0), (urows1_v, mrows1_v, sem1))

    def fire(q):
        urows_v, mrows_v, sem = bufs[q % 2]
        return (pltpu.async_copy(
                    packed_hbm.at[uidx_v.at[q]], urows_v, sem),
                pltpu.async_copy(
                    packed_hbm.at[midx_v.at[q]], mrows_v, sem))

    zero = jnp.zeros((LANES,), jnp.float32)
    accs = (zero, zero, zero, zero)
    inflight = [fire(0), fire(1)]
    for q in range(NCHUNK):
        urows_v, mrows_v, sem = bufs[q % 2]
        for cp in inflight[q]:
            cp.wait()

        def row_body(i, acc, urows_v=urows_v, mrows_v=mrows_v):
            out = []
            for j in range(E // LANES):
                usl = pl.ds(j * LANES, LANES)
                msl = pl.ds(E + j * LANES, LANES)
                out.append(acc[j] + urows_v[i, usl] * mrows_v[i, msl])
            return tuple(out)

        accs = lax.fori_loop(0, CHUNK, row_body, accs)
        if q + 2 < NCHUNK:
            inflight.append(fire(q + 2))

    pacc_v[...] = (accs[0] + accs[1]) + (accs[2] + accs[3])
    pltpu.sync_copy(pacc_v, partials_hbm.at[wid])

    for cp in bias_copies:
        cp.wait()
    for k in range(BPW // LANES):
        sl = pl.ds(k * LANES, LANES)
        bsum_v[sl] = ub_v[sl] + mb_v[sl]
    pltpu.sync_copy(bsum_v, bsum_hbm.at[pl.ds(base, BPW)])


def _sc_call(uidx, midx, packed, bias_both):
    return pl.kernel(
        _sc_body,
        out_type=(
            jax.ShapeDtypeStruct((NW, LANES), jnp.float32),
            jax.ShapeDtypeStruct((B,), jnp.float32),
        ),
        mesh=plsc.VectorSubcoreMesh(**_SC_MESH),
        compiler_params=_SC_PARAMS,
        scratch_types=[
            pltpu.VMEM((NCHUNK, CHUNK), jnp.int32),
            pltpu.VMEM((NCHUNK, CHUNK), jnp.int32),
            pltpu.VMEM((NCHUNK, CHUNK), jnp.int32),
            pltpu.VMEM((CHUNK, 2 * E), jnp.float32),
            pltpu.VMEM((CHUNK, 2 * E), jnp.float32),
            pltpu.VMEM((CHUNK, 2 * E), jnp.float32),
            pltpu.VMEM((CHUNK, 2 * E), jnp.float32),
            pltpu.VMEM((BPW,), jnp.float32),
            pltpu.VMEM((BPW,), jnp.float32),
            pltpu.VMEM((BPW,), jnp.float32),
            pltpu.VMEM((LANES,), jnp.float32),
            pltpu.SemaphoreType.DMA,
            pltpu.SemaphoreType.DMA,
            pltpu.SemaphoreType.DMA,
        ],
    )(uidx, midx, packed, bias_both)


def _tc_body(partials_ref, bsum_ref, out_ref):
    s = jnp.sum(partials_ref[...])
    out_ref[...] = jax.nn.sigmoid(bsum_ref[...] + s)


def _tc_call(partials, bsum2d):
    return pl.pallas_call(
        _tc_body,
        out_shape=jax.ShapeDtypeStruct(bsum2d.shape, jnp.float32),
    )(partials, bsum2d)


def kernel(inputs, user_emb, user_bias_tab, movie_emb, movie_bias_tab):
    uidx = inputs[:, 0].reshape(NW * NCHUNK, CHUNK)
    midx = inputs[:, 1].reshape(NW * NCHUNK, CHUNK)
    bias_both = jnp.concatenate(
        [user_bias_tab.reshape(-1), movie_bias_tab.reshape(-1)])
    packed = _pack_call(user_emb.T, movie_emb.T)
    partials, bsum = _sc_call(uidx, midx, packed, bias_both)
    y = _tc_call(partials, bsum.reshape(128, 128))
    return y.reshape(B, 1)


# speedup vs baseline: 1.0017x; 1.0017x over previous
import jax
import jax.numpy as jnp
from jax import lax
from jax.experimental import pallas as pl
from jax.experimental.pallas import tpu as pltpu
from jax.experimental.pallas import tpu_sc as plsc

NUM_CORES = 2
NUM_SUBCORES = 16
LANES = 16
NW = NUM_CORES * NUM_SUBCORES
B = 16384
E = 64
N = 100000
PACKC = 16384
BPW = B // NW
CHUNK = 128
NCHUNK = BPW // CHUNK

_SC_MESH = dict(core_axis_name="c", subcore_axis_name="s",
                num_cores=NUM_CORES, num_subcores=NUM_SUBCORES)
_SC_PARAMS = pltpu.CompilerParams(use_tc_tiling_on_sc=False)


def _pack_body(ut_ref, mt_ref, out_ref):
    x = jnp.concatenate([ut_ref[...], mt_ref[...]], axis=0)
    ii = lax.broadcasted_iota(jnp.int32, (2 * E, 2 * E), 0)
    jj = lax.broadcasted_iota(jnp.int32, (2 * E, 2 * E), 1)
    eye = (ii == jj).astype(jnp.float32)
    out_ref[...] = lax.dot_general(
        x, eye, dimension_numbers=(((0,), (0,)), ((), ())),
        preferred_element_type=jnp.float32)


def _pack_call(uembT, membT):
    grid = (N + PACKC - 1) // PACKC
    return pl.pallas_call(
        _pack_body,
        grid=(grid,),
        in_specs=[
            pl.BlockSpec((E, PACKC), lambda i: (0, i)),
            pl.BlockSpec((E, PACKC), lambda i: (0, i)),
        ],
        out_specs=pl.BlockSpec((PACKC, 2 * E), lambda i: (i, 0)),
        out_shape=jax.ShapeDtypeStruct((N, 2 * E), jnp.float32),
    )(uembT, membT)


def _sc_body(uidx_hbm, midx_hbm, packed_hbm, bias_hbm,
             partials_hbm, bsum_hbm,
             uidx_v, midx_v, midxb_v, urows0_v, mrows0_v, urows1_v, mrows1_v,
             ub_v, mb_v, bsum_v, pacc_v,
             sem0, sem1, bias_sem):
    wid = lax.axis_index("s") * NUM_CORES + lax.axis_index("c")
    base = wid * BPW

    pltpu.sync_copy(uidx_hbm.at[pl.ds(wid * NCHUNK, NCHUNK)], uidx_v)
    pltpu.sync_copy(midx_hbm.at[pl.ds(wid * NCHUNK, NCHUNK)], midx_v)

    for j in range(NCHUNK):
        for hh in range(CHUNK // LANES):
            sl = pl.ds(hh * LANES, LANES)
            midxb_v[j, sl] = midx_v[j, sl] + N
    bias_copies = []
    for j in range(NCHUNK):
        sl = pl.ds(j * CHUNK, CHUNK)
        bias_copies.append(pltpu.async_copy(
            bias_hbm.at[uidx_v.at[j]], ub_v.at[sl], bias_sem))
        bias_copies.append(pltpu.async_copy(
            bias_hbm.at[midxb_v.at[j]], mb_v.at[sl], bias_sem))

    bufs = ((urows0_v, mrows0_v, sem0), (urows1_v, mrows1_v, sem1))

    def fire(q):
        urows_v, mrows_v, sem = bufs[q % 2]
        return (pltpu.async_copy(
                    packed_hbm.at[uidx_v.at[q]], urows_v, sem),
                pltpu.async_copy(
                    packed_hbm.at[midx_v.at[q]], mrows_v, sem))

    zero = jnp.zeros((LANES,), jnp.float32)
    accs = (zero, zero, zero, zero)
    inflight = [fire(0), fire(1)]
    for q in range(NCHUNK):
        urows_v, mrows_v, sem = bufs[q % 2]
        for cp in inflight[q]:
            cp.wait()

        def row_body(i, acc, urows_v=urows_v, mrows_v=mrows_v):
            out = []
            for j in range(E // LANES):
                usl = pl.ds(j * LANES, LANES)
                msl = pl.ds(E + j * LANES, LANES)
                out.append(acc[j] + urows_v[i, usl] * mrows_v[i, msl])
            return tuple(out)

        accs = lax.fori_loop(0, CHUNK, row_body, accs)
        if q + 2 < NCHUNK:
            inflight.append(fire(q + 2))

    pacc_v[...] = (accs[0] + accs[1]) + (accs[2] + accs[3])
    pltpu.sync_copy(pacc_v, partials_hbm.at[wid])

    for cp in bias_copies:
        cp.wait()
    for k in range(BPW // LANES):
        sl = pl.ds(k * LANES, LANES)
        bsum_v[sl] = ub_v[sl] + mb_v[sl]
    pltpu.sync_copy(bsum_v, bsum_hbm.at[pl.ds(base, BPW)])


def _sc_call(uidx, midx, packed, bias_both):
    return pl.kernel(
        _sc_body,
        out_type=(
            jax.ShapeDtypeStruct((NW, LANES), jnp.float32),
            jax.ShapeDtypeStruct((B,), jnp.float32),
        ),
        mesh=plsc.VectorSubcoreMesh(**_SC_MESH),
        compiler_params=_SC_PARAMS,
        scratch_types=[
            pltpu.VMEM((NCHUNK, CHUNK), jnp.int32),
            pltpu.VMEM((NCHUNK, CHUNK), jnp.int32),
            pltpu.VMEM((NCHUNK, CHUNK), jnp.int32),
            pltpu.VMEM((CHUNK, 2 * E), jnp.float32),
            pltpu.VMEM((CHUNK, 2 * E), jnp.float32),
            pltpu.VMEM((CHUNK, 2 * E), jnp.float32),
            pltpu.VMEM((CHUNK, 2 * E), jnp.float32),
            pltpu.VMEM((BPW,), jnp.float32),
            pltpu.VMEM((BPW,), jnp.float32),
            pltpu.VMEM((BPW,), jnp.float32),
            pltpu.VMEM((LANES,), jnp.float32),
            pltpu.SemaphoreType.DMA,
            pltpu.SemaphoreType.DMA,
            pltpu.SemaphoreType.DMA,
        ],
    )(uidx, midx, packed, bias_both)


def _tc_body(partials_ref, bsum_ref, out_ref):
    s = jnp.sum(partials_ref[...])
    out_ref[...] = jax.nn.sigmoid(bsum_ref[...] + s)


def _tc_call(partials, bsum2d):
    return pl.pallas_call(
        _tc_body,
        out_shape=jax.ShapeDtypeStruct(bsum2d.shape, jnp.float32),
    )(partials, bsum2d)


def kernel(inputs, user_emb, user_bias_tab, movie_emb, movie_bias_tab):
    uidx = inputs[:, 0].reshape(NW * NCHUNK, CHUNK)
    midx = inputs[:, 1].reshape(NW * NCHUNK, CHUNK)
    bias_both = jnp.concatenate(
        [user_bias_tab.reshape(-1), movie_bias_tab.reshape(-1)])
    packed = _pack_call(user_emb.T, movie_emb.T)
    partials, bsum = _sc_call(uidx, midx, packed, bias_both)
    y = _tc_call(partials, bsum.reshape(128, 128))
    return y.reshape(B, 1)
